# baseline (device time: 79812 ns/iter reference)
import jax
import jax.numpy as jnp
from jax import lax
from jax.experimental import pallas as pl
from jax.experimental.pallas import tpu as pltpu

M, N = 2048, 1024
N_ROUNDS = 5
N_PARTS = 2
PM = M // N_PARTS
HALF = [512, 256, 128, 64, 32]
COMM_OFF = [0, 512, 768, 896, 960]
COMM_ROWS = 992


def _coords(d):
    z = d // 8
    s8 = d % 8
    x = ((s8 + 1) >> 1) & 1
    y = s8 >> 1
    return x, y, z


def _logical_id(x, y, z):
    return 8 * z + 2 * y + (x ^ (y & 1))


def _mo(off):
    return pl.multiple_of(off, 32)


def kernel(t):
    def body(x_ref, out_ref, comm_ref, stage_ref, g_ref, send_sems, recv_sems):
        d = lax.axis_index("i")
        x, y, z = _coords(d)

        dim_x = (_logical_id(1 - x, y, z), x)
        dim_y0 = (_logical_id(x, y ^ 1, z), y & 1)
        dim_y1 = (_logical_id(x, y ^ 2, z), (y >> 1) & 1)
        dim_z0 = (_logical_id(x, y, z ^ 1), z & 1)
        dim_z1 = (_logical_id(x, y, z ^ 2), (z >> 1) & 1)
        parts = [
            [dim_x, dim_y0, dim_z0, dim_z1, dim_y1],
            [dim_y0, dim_z0, dim_x, dim_y1, dim_z1],
        ]

        barrier_sem = pltpu.get_barrier_semaphore()
        for p, _ in parts[0]:
            pl.semaphore_signal(
                barrier_sem, inc=1,
                device_id=(p,), device_id_type=pl.DeviceIdType.MESH,
            )
        pl.semaphore_wait(barrier_sem, N_ROUNDS)


        def sem_idx(part, r, ag):
            return (2 * part + ag) * N_ROUNDS + r

        def rs_rdma(part, r, o):
            partner, bit = parts[part][r]
            half = HALF[r]
            send_off = o + jnp.where(bit == 0, half, 0)
            keep_off = o + jnp.where(bit == 0, 0, half)
            c = part * COMM_ROWS + COMM_OFF[r]
            acc = x_ref if r == 0 else out_ref
            stage_ref[pl.ds(c, half), :] = acc[
                pl.ds(_mo(send_off), half), :
            ].astype(jnp.bfloat16)
            rdma = pltpu.make_async_remote_copy(
                src_ref=stage_ref.at[pl.ds(c, half), :],
                dst_ref=comm_ref.at[pl.ds(c, half), :],
                send_sem=send_sems.at[sem_idx(part, r, 0)],
                recv_sem=recv_sems.at[sem_idx(part, r, 0)],
                device_id=(partner,),
                device_id_type=pl.DeviceIdType.MESH,
            )
            rdma.start()
            return rdma, keep_off

        def rs_accum(part, r, keep_off):
            half = HALF[r]
            c = part * COMM_ROWS + COMM_OFF[r]
            acc = x_ref if r == 0 else out_ref
            out_ref[pl.ds(_mo(keep_off), half), :] = (
                acc[pl.ds(_mo(keep_off), half), :]
                + comm_ref[pl.ds(c, half), :].astype(jnp.float32)
            )

        offs = [jnp.int32(part * PM) for part in range(N_PARTS)]
        for r in range(N_ROUNDS):
            started = []
            for part in range(N_PARTS):
                rdma, keep_off = rs_rdma(part, r, offs[part])
                started.append(rdma)
                offs[part] = keep_off
            for part in range(N_PARTS):
                started[part].wait()
                rs_accum(part, r, offs[part])

        for part in range(N_PARTS):
            s = out_ref[pl.ds(_mo(offs[part]), 32), :]
            rel = jnp.maximum(s, 0.0)
            fs = jnp.tanh(s) * s * s + rel * rel * rel
            out_ref[pl.ds(_mo(offs[part]), 32), :] = fs
            g_ref[pl.ds(_mo(offs[part]), 32), :] = fs.astype(jnp.bfloat16)

        sz = PM // 32
        for r in reversed(range(N_ROUNDS)):
            started = []
            for part in range(N_PARTS):
                partner, _ = parts[part][r]
                rdma = pltpu.make_async_remote_copy(
                    src_ref=g_ref.at[pl.ds(_mo(offs[part]), sz), :],
                    dst_ref=g_ref.at[pl.ds(_mo(offs[part]), sz), :],
                    send_sem=send_sems.at[sem_idx(part, r, 1)],
                    recv_sem=recv_sems.at[sem_idx(part, r, 1)],
                    device_id=(partner,),
                    device_id_type=pl.DeviceIdType.MESH,
                )
                rdma.start()
                started.append(rdma)
            for part in range(N_PARTS):
                _, bit = parts[part][r]
                started[part].wait()
                parent = offs[part] - bit * sz
                recv_off = 2 * parent + sz - offs[part]
                out_ref[pl.ds(_mo(recv_off), sz), :] = g_ref[
                    pl.ds(_mo(recv_off), sz), :
                ].astype(jnp.float32)
                offs[part] = parent
            sz *= 2

    return pl.pallas_call(
        body,
        out_shape=jax.ShapeDtypeStruct((M, N), jnp.float32),
        in_specs=[pl.BlockSpec(memory_space=pltpu.VMEM)],
        out_specs=pl.BlockSpec(memory_space=pltpu.VMEM),
        scratch_shapes=[
            pltpu.VMEM((N_PARTS * COMM_ROWS, N), jnp.bfloat16),
            pltpu.VMEM((N_PARTS * COMM_ROWS, N), jnp.bfloat16),
            pltpu.VMEM((M, N), jnp.bfloat16),
            pltpu.SemaphoreType.DMA((4 * N_ROUNDS,)),
            pltpu.SemaphoreType.DMA((4 * N_ROUNDS,)),
        ],
        compiler_params=pltpu.CompilerParams(collective_id=0),
    )(t)


# device time: 77161 ns/iter; 1.0344x vs baseline; 1.0344x over previous
import jax
import jax.numpy as jnp
from jax import lax
from jax.experimental import pallas as pl
from jax.experimental.pallas import tpu as pltpu

M, N = 2048, 1024
N_BFLY = 3
N_PARTS = 2
PM = M // N_PARTS
HALF = [512, 256, 128]
COMM_OFF = [0, 512, 768]
GRP_OFF = 896
COMM_ROWS = 992
FLIPS = [(0, 1), (1, 0), (1, 1)]


def _coords(d):
    z = d // 8
    s8 = d % 8
    x = ((s8 + 1) >> 1) & 1
    y = s8 >> 1
    return x, y, z


def _logical_id(x, y, z):
    return 8 * z + 2 * y + (x ^ (y & 1))


def _mo(off):
    return pl.multiple_of(off, 32)


def kernel(t):
    def body(x_ref, out_ref, comm_ref, stage_ref, g_ref, send_sems, recv_sems):
        d = lax.axis_index("i")
        x, y, z = _coords(d)
        gy, gz = (y >> 1) & 1, (z >> 1) & 1

        dim_x = (_logical_id(1 - x, y, z), x)
        dim_y0 = (_logical_id(x, y ^ 1, z), y & 1)
        dim_z0 = (_logical_id(x, y, z ^ 1), z & 1)
        parts = [
            [dim_x, dim_y0, dim_z0],
            [dim_y0, dim_z0, dim_x],
        ]
        def grp_sub(part, fy, fz):
            wz, wy = (64, 32) if part == 0 else (32, 64)
            return (gz ^ fz) * wz + (gy ^ fy) * wy

        def grp_peer(fy, fz):
            return _logical_id(x, y ^ (2 * fy), z ^ (2 * fz))

        barrier_sem = pltpu.get_barrier_semaphore()
        barrier_peers = [p for p, _ in parts[0]] + [
            grp_peer(fy, fz) for fy, fz in FLIPS
        ]
        for p in barrier_peers:
            pl.semaphore_signal(
                barrier_sem, inc=1,
                device_id=(p,), device_id_type=pl.DeviceIdType.MESH,
            )
        pl.semaphore_wait(barrier_sem, len(barrier_peers))

        def bfly_sem(part, r, ag):
            return (0 if not ag else 18) + part * N_BFLY + r

        def grp_sem(part, k, ag):
            return (6 if not ag else 12) + part * 3 + k

        offs = [jnp.int32(part * PM) for part in range(N_PARTS)]
        for r in range(N_BFLY):
            half = HALF[r]
            acc = x_ref if r == 0 else out_ref
            started = []
            for part in range(N_PARTS):
                partner, bit = parts[part][r]
                o = offs[part]
                send_off = o + jnp.where(bit == 0, half, 0)
                keep_off = o + jnp.where(bit == 0, 0, half)
                c = part * COMM_ROWS + COMM_OFF[r]
                stage_ref[pl.ds(c, half), :] = acc[
                    pl.ds(_mo(send_off), half), :
                ].astype(jnp.bfloat16)
                rdma = pltpu.make_async_remote_copy(
                    src_ref=stage_ref.at[pl.ds(c, half), :],
                    dst_ref=comm_ref.at[pl.ds(c, half), :],
                    send_sem=send_sems.at[bfly_sem(part, r, 0)],
                    recv_sem=recv_sems.at[bfly_sem(part, r, 0)],
                    device_id=(partner,),
                    device_id_type=pl.DeviceIdType.MESH,
                )
                rdma.start()
                started.append(rdma)
                offs[part] = keep_off
            for part in range(N_PARTS):
                started[part].wait()
                c = part * COMM_ROWS + COMM_OFF[r]
                out_ref[pl.ds(_mo(offs[part]), half), :] = (
                    acc[pl.ds(_mo(offs[part]), half), :]
                    + comm_ref[pl.ds(c, half), :].astype(jnp.float32)
                )

        rs_grp = []
        for part in range(N_PARTS):
            for k, (fy, fz) in enumerate(FLIPS):
                c = part * COMM_ROWS + GRP_OFF + 32 * k
                peer_off = offs[part] + grp_sub(part, fy, fz)
                stage_ref[pl.ds(c, 32), :] = out_ref[
                    pl.ds(_mo(peer_off), 32), :
                ].astype(jnp.bfloat16)
                rdma = pltpu.make_async_remote_copy(
                    src_ref=stage_ref.at[pl.ds(c, 32), :],
                    dst_ref=comm_ref.at[pl.ds(c, 32), :],
                    send_sem=send_sems.at[grp_sem(part, k, 0)],
                    recv_sem=recv_sems.at[grp_sem(part, k, 0)],
                    device_id=(grp_peer(fy, fz),),
                    device_id_type=pl.DeviceIdType.MESH,
                )
                rdma.start()
                rs_grp.append(rdma)

        my_offs = []
        ag_grp = []
        for part in range(N_PARTS):
            for k in range(3):
                rs_grp[part * 3 + k].wait()
            my_off = offs[part] + grp_sub(part, 0, 0)
            my_offs.append(my_off)
            cbase = part * COMM_ROWS + GRP_OFF
            s = (
                out_ref[pl.ds(_mo(my_off), 32), :]
                + comm_ref[pl.ds(cbase, 32), :].astype(jnp.float32)
                + comm_ref[pl.ds(cbase + 32, 32), :].astype(jnp.float32)
                + comm_ref[pl.ds(cbase + 64, 32), :].astype(jnp.float32)
            )
            rel = jnp.maximum(s, 0.0)
            fs = jnp.tanh(s) * s * s + rel * rel * rel
            out_ref[pl.ds(_mo(my_off), 32), :] = fs
            g_ref[pl.ds(_mo(my_off), 32), :] = fs.astype(jnp.bfloat16)
            for k, (fy, fz) in enumerate(FLIPS):
                rdma = pltpu.make_async_remote_copy(
                    src_ref=g_ref.at[pl.ds(_mo(my_off), 32), :],
                    dst_ref=g_ref.at[pl.ds(_mo(my_off), 32), :],
                    send_sem=send_sems.at[grp_sem(part, k, 1)],
                    recv_sem=recv_sems.at[grp_sem(part, k, 1)],
                    device_id=(grp_peer(fy, fz),),
                    device_id_type=pl.DeviceIdType.MESH,
                )
                rdma.start()
                ag_grp.append(rdma)

        pend = []
        for part in range(N_PARTS):
            for k in range(3):
                ag_grp[part * 3 + k].wait()
            for fy, fz in FLIPS:
                pend.append((offs[part] + grp_sub(part, fy, fz), 32))

        sz = 4 * 32
        for r in reversed(range(N_BFLY)):
            started = []
            for part in range(N_PARTS):
                partner, _ = parts[part][r]
                rdma = pltpu.make_async_remote_copy(
                    src_ref=g_ref.at[pl.ds(_mo(offs[part]), sz), :],
                    dst_ref=g_ref.at[pl.ds(_mo(offs[part]), sz), :],
                    send_sem=send_sems.at[bfly_sem(part, r, 1)],
                    recv_sem=recv_sems.at[bfly_sem(part, r, 1)],
                    device_id=(partner,),
                    device_id_type=pl.DeviceIdType.MESH,
                )
                rdma.start()
                started.append(rdma)
            for off, n_rows in pend:
                out_ref[pl.ds(_mo(off), n_rows), :] = g_ref[
                    pl.ds(_mo(off), n_rows), :
                ].astype(jnp.float32)
            pend = []
            for part in range(N_PARTS):
                _, bit = parts[part][r]
                started[part].wait()
                parent = offs[part] - bit * sz
                pend.append((2 * parent + sz - offs[part], sz))
                offs[part] = parent
            sz *= 2
        for off, n_rows in pend:
            out_ref[pl.ds(_mo(off), n_rows), :] = g_ref[
                pl.ds(_mo(off), n_rows), :
            ].astype(jnp.float32)

    return pl.pallas_call(
        body,
        out_shape=jax.ShapeDtypeStruct((M, N), jnp.float32),
        in_specs=[pl.BlockSpec(memory_space=pltpu.VMEM)],
        out_specs=pl.BlockSpec(memory_space=pltpu.VMEM),
        scratch_shapes=[
            pltpu.VMEM((N_PARTS * COMM_ROWS, N), jnp.bfloat16),
            pltpu.VMEM((N_PARTS * COMM_ROWS, N), jnp.bfloat16),
            pltpu.VMEM((M, N), jnp.bfloat16),
            pltpu.SemaphoreType.DMA((24,)),
            pltpu.SemaphoreType.DMA((24,)),
        ],
        compiler_params=pltpu.CompilerParams(collective_id=0),
    )(t)


# device time: 76508 ns/iter; 1.0432x vs baseline; 1.0085x over previous
import jax
import jax.numpy as jnp
from jax import lax
from jax.experimental import pallas as pl
from jax.experimental.pallas import tpu as pltpu

M, N = 2048, 1024
N_BFLY = 3
N_PARTS = 2
PM = M // N_PARTS
HALF = [512, 256, 128]
COMM_OFF = [0, 512, 768]
GRP_OFF = 896
COMM_ROWS = 992
FLIPS = [(0, 1), (1, 0), (1, 1)]


def _coords(d):
    z = d // 8
    s8 = d % 8
    x = ((s8 + 1) >> 1) & 1
    y = s8 >> 1
    return x, y, z


def _logical_id(x, y, z):
    return 8 * z + 2 * y + (x ^ (y & 1))


def _mo(off):
    return pl.multiple_of(off, 32)


def kernel(t):
    def body(x_ref, out_ref, comm_ref, stage_ref, g_ref, send_sems, recv_sems):
        d = lax.axis_index("i")
        x, y, z = _coords(d)
        gy, gz = (y >> 1) & 1, (z >> 1) & 1

        dim_x = (_logical_id(1 - x, y, z), x)
        dim_y0 = (_logical_id(x, y ^ 1, z), y & 1)
        dim_z0 = (_logical_id(x, y, z ^ 1), z & 1)
        parts = [
            [dim_x, dim_y0, dim_z0],
            [dim_y0, dim_z0, dim_x],
        ]

        def grp_sub(part, fy, fz):
            wz, wy = (64, 32) if part == 0 else (32, 64)
            return (gz ^ fz) * wz + (gy ^ fy) * wy

        def grp_peer(fy, fz):
            return _logical_id(x, y ^ (2 * fy), z ^ (2 * fz))

        barrier_sem = pltpu.get_barrier_semaphore()
        barrier_peers = [p for p, _ in parts[0]] + [
            grp_peer(fy, fz) for fy, fz in FLIPS
        ]
        for p in barrier_peers:
            pl.semaphore_signal(
                barrier_sem, inc=1,
                device_id=(p,), device_id_type=pl.DeviceIdType.MESH,
            )
        pl.semaphore_wait(barrier_sem, len(barrier_peers))

        def rs_bfly_sem(part, r):
            return part * 3 + r

        def rs_grp_sem(part, k):
            return 6 + part * 3 + k

        def ag_grp_sem(part, k):
            return 12 + part * 3 + k

        def _rdma(src, dst, sem, peer):
            rdma = pltpu.make_async_remote_copy(
                src_ref=src,
                dst_ref=dst,
                send_sem=send_sems.at[sem],
                recv_sem=recv_sems.at[sem],
                device_id=(peer,),
                device_id_type=pl.DeviceIdType.MESH,
            )
            rdma.start()
            return rdma

        def rs_start(part, r, o, acc):
            partner, bit = parts[part][r]
            half = HALF[r]
            send_off = o + jnp.where(bit == 0, half, 0)
            keep_off = o + jnp.where(bit == 0, 0, half)
            c = part * COMM_ROWS + COMM_OFF[r]
            stage_ref[pl.ds(c, half), :] = acc[
                pl.ds(_mo(send_off), half), :
            ].astype(jnp.bfloat16)
            rdma = _rdma(
                stage_ref.at[pl.ds(c, half), :],
                comm_ref.at[pl.ds(c, half), :],
                rs_bfly_sem(part, r),
                partner,
            )
            return rdma, keep_off

        def rs_accum(part, r, keep_off, sub_off, n_rows, acc):
            c = part * COMM_ROWS + COMM_OFF[r] + (sub_off - keep_off)
            out_ref[pl.ds(_mo(sub_off), n_rows), :] = (
                acc[pl.ds(_mo(sub_off), n_rows), :]
                + comm_ref[pl.ds(_mo(c), n_rows), :].astype(jnp.float32)
            )

        offs = [jnp.int32(part * PM) for part in range(N_PARTS)]
        rs_inflight = []
        for part in range(N_PARTS):
            rdma, keep_off = rs_start(part, 0, offs[part], x_ref)
            rs_inflight.append(rdma)
            offs[part] = keep_off

        rs_grp = [[], []]
        for r in range(N_BFLY):
            acc = x_ref if r == 0 else out_ref
            for part in range(N_PARTS):
                rs_inflight[part].wait()
                o = offs[part]
                half = HALF[r]
                if r + 1 < N_BFLY:
                    _, nbit = parts[part][r + 1]
                    nh = HALF[r + 1]
                    nsend = o + jnp.where(nbit == 0, nh, 0)
                    nkeep = o + jnp.where(nbit == 0, 0, nh)
                    rs_accum(part, r, o, nsend, nh, acc)
                    rdma, _ = rs_start(part, r + 1, o, out_ref)
                    rs_inflight[part] = rdma
                    rs_accum(part, r, o, nkeep, nh, acc)
                    offs[part] = nkeep
                else:
                    for k, (fy, fz) in enumerate(FLIPS):
                        poff = o + grp_sub(part, fy, fz)
                        rs_accum(part, r, o, poff, 32, acc)
                        c = part * COMM_ROWS + GRP_OFF + 32 * k
                        stage_ref[pl.ds(c, 32), :] = out_ref[
                            pl.ds(_mo(poff), 32), :
                        ].astype(jnp.bfloat16)
                        rs_grp[part].append(
                            _rdma(
                                stage_ref.at[pl.ds(c, 32), :],
                                comm_ref.at[pl.ds(c, 32), :],
                                rs_grp_sem(part, k),
                                grp_peer(fy, fz),
                            )
                        )
                    my_off = o + grp_sub(part, 0, 0)
                    rs_accum(part, r, o, my_off, 32, acc)

        ag_grp = [[], []]
        for part in range(N_PARTS):
            for k in range(3):
                rs_grp[part][k].wait()
            my_off = offs[part] + grp_sub(part, 0, 0)
            cbase = part * COMM_ROWS + GRP_OFF
            s = (
                out_ref[pl.ds(_mo(my_off), 32), :]
                + comm_ref[pl.ds(cbase, 32), :].astype(jnp.float32)
                + comm_ref[pl.ds(cbase + 32, 32), :].astype(jnp.float32)
                + comm_ref[pl.ds(cbase + 64, 32), :].astype(jnp.float32)
            )
            rel = jnp.maximum(s, 0.0)
            fs = jnp.tanh(s) * s * s + rel * rel * rel
            out_ref[pl.ds(_mo(my_off), 32), :] = fs
            g_ref[pl.ds(_mo(my_off), 32), :] = fs.astype(jnp.bfloat16)
            for k, (fy, fz) in enumerate(FLIPS):
                ag_grp[part].append(
                    _rdma(
                        g_ref.at[pl.ds(_mo(my_off), 32), :],
                        g_ref.at[pl.ds(_mo(my_off), 32), :],
                        ag_grp_sem(part, k),
                        grp_peer(fy, fz),
                    )
                )

        def convert(off, n_rows):
            out_ref[pl.ds(_mo(off), n_rows), :] = g_ref[
                pl.ds(_mo(off), n_rows), :
            ].astype(jnp.float32)

        def ag_start(part, r, o, sz):
            partner, _ = parts[part][r]
            if r > 0:
                return [
                    _rdma(
                        g_ref.at[pl.ds(_mo(o), sz), :],
                        g_ref.at[pl.ds(_mo(o), sz), :],
                        18 + 2 * (2 - r) + part,
                        partner,
                    )
                ]
            h = sz // 2
            return [
                _rdma(
                    g_ref.at[pl.ds(_mo(o + i * h), h), :],
                    g_ref.at[pl.ds(_mo(o + i * h), h), :],
                    22 + 2 * i + part,
                    partner,
                )
                for i in range(2)
            ]

        pend = []
        ag_inflight = []
        for part in range(N_PARTS):
            for k in range(3):
                ag_grp[part][k].wait()
            ag_inflight.append(ag_start(part, N_BFLY - 1, offs[part], 4 * 32))
            for fy, fz in FLIPS:
                pend.append((offs[part] + grp_sub(part, fy, fz), 32))

        sz = 4 * 32
        for r in reversed(range(N_BFLY)):
            for off, n_rows in pend:
                convert(off, n_rows)
            pend = []
            for part in range(N_PARTS):
                _, bit = parts[part][r]
                parent = offs[part] - bit * sz
                recv_off = 2 * parent + sz - offs[part]
                offs[part] = parent
                if r > 0:
                    ag_inflight[part][0].wait()
                    ag_inflight[part] = ag_start(part, r - 1, parent, 2 * sz)
                    pend.append((recv_off, sz))
                else:
                    h = sz // 2
                    ag_inflight[part][0].wait()
                    convert(recv_off, h)
                    ag_inflight[part][1].wait()
                    convert(recv_off + h, h)
            sz *= 2

    return pl.pallas_call(
        body,
        out_shape=jax.ShapeDtypeStruct((M, N), jnp.float32),
        in_specs=[pl.BlockSpec(memory_space=pltpu.VMEM)],
        out_specs=pl.BlockSpec(memory_space=pltpu.VMEM),
        scratch_shapes=[
            pltpu.VMEM((N_PARTS * COMM_ROWS, N), jnp.bfloat16),
            pltpu.VMEM((N_PARTS * COMM_ROWS, N), jnp.bfloat16),
            pltpu.VMEM((M, N), jnp.bfloat16),
            pltpu.SemaphoreType.DMA((26,)),
            pltpu.SemaphoreType.DMA((26,)),
        ],
        compiler_params=pltpu.CompilerParams(collective_id=0),
    )(t)


# device time: 76490 ns/iter; 1.0434x vs baseline; 1.0002x over previous
import contextlib
import os

import jax
import jax.numpy as jnp
from jax import lax
from jax.experimental import pallas as pl
from jax.experimental.pallas import tpu as pltpu

M, N = 2048, 1024
N_BFLY = 3
N_PARTS = 2
PM = M // N_PARTS
HALF = [512, 256, 128]
COMM_OFF = [0, 512, 768]
GRP_OFF = 896
COMM_ROWS = 992
FLIPS = [(0, 1), (1, 0), (1, 1)]


def _coords(d):
    z = d // 8
    s8 = d % 8
    x = ((s8 + 1) >> 1) & 1
    y = s8 >> 1
    return x, y, z


def _logical_id(x, y, z):
    return 8 * z + 2 * y + (x ^ (y & 1))


def _mo(off):
    return pl.multiple_of(off, 32)


_PROFILE = os.environ.get("PROFILE_SCOPES") == "1"


def _scope(name):
    return jax.named_scope(name) if _PROFILE else contextlib.nullcontext()


def kernel(t):
    def body(x_ref, out_ref, comm_ref, stage_ref, g_ref, send_sems, recv_sems):
        d = lax.axis_index("i")
        x, y, z = _coords(d)
        gy, gz = (y >> 1) & 1, (z >> 1) & 1

        dim_x = (_logical_id(1 - x, y, z), x)
        dim_y0 = (_logical_id(x, y ^ 1, z), y & 1)
        dim_z0 = (_logical_id(x, y, z ^ 1), z & 1)
        parts = [
            [dim_x, dim_y0, dim_z0],
            [dim_y0, dim_z0, dim_x],
        ]

        def grp_sub(part, fy, fz):
            wz, wy = (64, 32) if part == 0 else (32, 64)
            return (gz ^ fz) * wz + (gy ^ fy) * wy

        def grp_peer(fy, fz):
            return _logical_id(x, y ^ (2 * fy), z ^ (2 * fz))

        ctx = contextlib.ExitStack()
        ctx.enter_context(_scope("barrier"))
        barrier_sem = pltpu.get_barrier_semaphore()
        barrier_peers = [p for p, _ in parts[0]] + [
            grp_peer(fy, fz) for fy, fz in FLIPS
        ]
        for p in barrier_peers:
            pl.semaphore_signal(
                barrier_sem, inc=1,
                device_id=(p,), device_id_type=pl.DeviceIdType.MESH,
            )
        pl.semaphore_wait(barrier_sem, len(barrier_peers))

        def rs_bfly_sem(part, r):
            return part * 3 + r

        def rs_grp_sem(part, k):
            return 6 + part * 3 + k

        def ag_grp_sem(part, k):
            return 12 + part * 3 + k

        def _rdma(src, dst, sem, peer):
            rdma = pltpu.make_async_remote_copy(
                src_ref=src,
                dst_ref=dst,
                send_sem=send_sems.at[sem],
                recv_sem=recv_sems.at[sem],
                device_id=(peer,),
                device_id_type=pl.DeviceIdType.MESH,
            )
            rdma.start()
            return rdma

        def rs_start(part, r, o, acc):
            partner, bit = parts[part][r]
            half = HALF[r]
            send_off = o + jnp.where(bit == 0, half, 0)
            keep_off = o + jnp.where(bit == 0, 0, half)
            c = part * COMM_ROWS + COMM_OFF[r]
            stage_ref[pl.ds(c, half), :] = acc[
                pl.ds(_mo(send_off), half), :
            ].astype(jnp.bfloat16)
            rdma = _rdma(
                stage_ref.at[pl.ds(c, half), :],
                comm_ref.at[pl.ds(c, half), :],
                rs_bfly_sem(part, r),
                partner,
            )
            return rdma, keep_off

        def rs_accum(part, r, keep_off, sub_off, n_rows, acc):
            c = part * COMM_ROWS + COMM_OFF[r] + (sub_off - keep_off)
            out_ref[pl.ds(_mo(sub_off), n_rows), :] = (
                acc[pl.ds(_mo(sub_off), n_rows), :]
                + comm_ref[pl.ds(_mo(c), n_rows), :].astype(jnp.float32)
            )

        ctx.close()
        offs = [jnp.int32(part * PM) for part in range(N_PARTS)]
        rs_inflight = []
        for part in range(N_PARTS):
            rdma, keep_off = rs_start(part, 0, offs[part], x_ref)
            rs_inflight.append(rdma)
            offs[part] = keep_off

        rs_grp = [[], []]
        for r in range(N_BFLY):
            rctx = contextlib.ExitStack()
            rctx.enter_context(_scope(f"rs_r{r}"))
            acc = x_ref if r == 0 else out_ref
            for part in range(N_PARTS):
                rs_inflight[part].wait()
                o = offs[part]
                half = HALF[r]
                if r + 1 < N_BFLY:
                    _, nbit = parts[part][r + 1]
                    nh = HALF[r + 1]
                    nsend = o + jnp.where(nbit == 0, nh, 0)
                    nkeep = o + jnp.where(nbit == 0, 0, nh)
                    rs_accum(part, r, o, nsend, nh, acc)
                    rdma, _ = rs_start(part, r + 1, o, out_ref)
                    rs_inflight[part] = rdma
                    rs_accum(part, r, o, nkeep, nh, acc)
                    offs[part] = nkeep
                else:
                    for k, (fy, fz) in enumerate(FLIPS):
                        poff = o + grp_sub(part, fy, fz)
                        rs_accum(part, r, o, poff, 32, acc)
                        c = part * COMM_ROWS + GRP_OFF + 32 * k
                        stage_ref[pl.ds(c, 32), :] = out_ref[
                            pl.ds(_mo(poff), 32), :
                        ].astype(jnp.bfloat16)
                        rs_grp[part].append(
                            _rdma(
                                stage_ref.at[pl.ds(c, 32), :],
                                comm_ref.at[pl.ds(c, 32), :],
                                rs_grp_sem(part, k),
                                grp_peer(fy, fz),
                            )
                        )
                    my_off = o + grp_sub(part, 0, 0)
                    rs_accum(part, r, o, my_off, 32, acc)
            rctx.close()

        ag_grp = [[], []]
        gctx = contextlib.ExitStack()
        gctx.enter_context(_scope("grp_reduce_f"))
        for part in range(N_PARTS):
            for k in range(3):
                rs_grp[part][k].wait()
            my_off = offs[part] + grp_sub(part, 0, 0)
            cbase = part * COMM_ROWS + GRP_OFF
            s = (
                out_ref[pl.ds(_mo(my_off), 32), :]
                + comm_ref[pl.ds(cbase, 32), :].astype(jnp.float32)
                + comm_ref[pl.ds(cbase + 32, 32), :].astype(jnp.float32)
                + comm_ref[pl.ds(cbase + 64, 32), :].astype(jnp.float32)
            )
            rel = jnp.maximum(s, 0.0)
            fs = jnp.tanh(s) * s * s + rel * rel * rel
            out_ref[pl.ds(_mo(my_off), 32), :] = fs
            g_ref[pl.ds(_mo(my_off), 32), :] = fs.astype(jnp.bfloat16)
            for k, (fy, fz) in enumerate(FLIPS):
                ag_grp[part].append(
                    _rdma(
                        g_ref.at[pl.ds(_mo(my_off), 32), :],
                        g_ref.at[pl.ds(_mo(my_off), 32), :],
                        ag_grp_sem(part, k),
                        grp_peer(fy, fz),
                    )
                )

        def convert(off, n_rows):
            out_ref[pl.ds(_mo(off), n_rows), :] = g_ref[
                pl.ds(_mo(off), n_rows), :
            ].astype(jnp.float32)

        def ag_start(part, r, o, sz):
            partner, _ = parts[part][r]
            if r > 0:
                return [
                    _rdma(
                        g_ref.at[pl.ds(_mo(o), sz), :],
                        g_ref.at[pl.ds(_mo(o), sz), :],
                        18 + 2 * (2 - r) + part,
                        partner,
                    )
                ]
            h = sz // 2
            return [
                _rdma(
                    g_ref.at[pl.ds(_mo(o + i * h), h), :],
                    g_ref.at[pl.ds(_mo(o + i * h), h), :],
                    22 + 2 * i + part,
                    partner,
                )
                for i in range(2)
            ]

        gctx.close()
        pend = []
        ag_inflight = []
        actx = contextlib.ExitStack()
        actx.enter_context(_scope("ag_grp"))
        for part in range(N_PARTS):
            for k in range(3):
                ag_grp[part][k].wait()
            ag_inflight.append(ag_start(part, N_BFLY - 1, offs[part], 4 * 32))
            for fy, fz in FLIPS:
                pend.append((offs[part] + grp_sub(part, fy, fz), 32))

        actx.close()
        sz = 4 * 32
        for r in reversed(range(N_BFLY)):
            rctx = contextlib.ExitStack()
            rctx.enter_context(_scope(f"ag_r{r}"))
            for off, n_rows in pend:
                convert(off, n_rows)
            pend = []
            for part in range(N_PARTS):
                _, bit = parts[part][r]
                parent = offs[part] - bit * sz
                recv_off = 2 * parent + sz - offs[part]
                offs[part] = parent
                if r > 0:
                    ag_inflight[part][0].wait()
                    ag_inflight[part] = ag_start(part, r - 1, parent, 2 * sz)
                    pend.append((recv_off, sz))
                else:
                    h = sz // 2
                    ag_inflight[part][0].wait()
                    convert(recv_off, h)
                    ag_inflight[part][1].wait()
                    convert(recv_off + h, h)
            rctx.close()
            sz *= 2

    return pl.pallas_call(
        body,
        out_shape=jax.ShapeDtypeStruct((M, N), jnp.float32),
        in_specs=[pl.BlockSpec(memory_space=pltpu.VMEM)],
        out_specs=pl.BlockSpec(memory_space=pltpu.VMEM),
        scratch_shapes=[
            pltpu.VMEM((N_PARTS * COMM_ROWS, N), jnp.bfloat16),
            pltpu.VMEM((N_PARTS * COMM_ROWS, N), jnp.bfloat16),
            pltpu.VMEM((M, N), jnp.bfloat16),
            pltpu.SemaphoreType.DMA((26,)),
            pltpu.SemaphoreType.DMA((26,)),
        ],
        compiler_params=pltpu.CompilerParams(collective_id=0),
    )(t)
